# baseline (speedup 1.0000x reference)
import jax
import jax.numpy as jnp
from jax import lax
from jax.experimental import pallas as pl
from jax.experimental.pallas import tpu as pltpu
from jax.experimental.pallas import tpu_sc as plsc

B = 16384
D = 32
K = 20
NC, NS, L = 2, 16, 16
NW = NC * NS
BPW = B // NW
V = 1000000
WIN = 512
NFULL = 1953
HG = 64
NHG = BPW // HG


def _conv_body(out_t, emb_t, tail_o, tail_e, conv_o, conv_e,
               winA, winB, tpA, tpB,
               semIA, semIB, semOA, semOB):
    wid = lax.axis_index("s") * NC + lax.axis_index("c")
    iota = lax.iota(jnp.int32, L)
    iota32 = iota * D
    for (src, tail16, conv) in ((out_t, tail_o, conv_o),
                                (emb_t, tail_e, conv_e)):
        _one_table(src, tail16, conv, wid, iota32,
                   winA, winB, tpA, tpB, semIA, semIB, semOA, semOB)


def _one_table(out_t, tail16, conv, wid, iota32,
               winA, winB, tpA, tpB, semIA, semIB, semOA, semOB):

    def win_src(t):
        return out_t.at[:, pl.ds(t * WIN, WIN)]

    def fire_in(t, buf, sem):
        pltpu.async_copy(win_src(t), buf, sem)

    def drain_in(buf, sem):
        pltpu.make_async_copy(win_src(0), buf, sem).wait()

    def transpose(buf, tp):
        def drow(d, _):
            for c in range(WIN // L):
                w = buf[d, pl.ds(c * L, L)]
                plsc.store_scatter(tp, [iota32 + (c * L * D + d)], w)
            return _
        lax.fori_loop(0, D, drow, 0)

    def fire_out(t, tp, sem):
        pltpu.async_copy(tp, conv.at[pl.ds(t * (WIN * D), WIN * D)], sem)

    def drain_out(tp, sem):
        pltpu.make_async_copy(conv.at[pl.ds(0, WIN * D)], tp, sem).wait()

    def tA(jj):
        return wid + 32 * (2 * jj)

    def tB(jj):
        return wid + 32 * (2 * jj + 1)

    fire_in(tA(0), winA, semIA)
    fire_in(tB(0), winB, semIB)

    def step(jj, cr):
        validB = jnp.logical_or(jj < 30, wid == 0)

        drain_in(winA, semIA)

        @pl.when(jj > 0)
        def _():
            drain_out(tpA, semOA)
        transpose(winA, tpA)
        fire_out(tA(jj), tpA, semOA)

        @pl.when(jj < 30)
        def _():
            fire_in(tA(jj + 1), winA, semIA)

        @pl.when(validB)
        def _():
            drain_in(winB, semIB)

            @pl.when(jj > 0)
            def _():
                drain_out(tpB, semOB)
            transpose(winB, tpB)
            fire_out(tB(jj), tpB, semOB)

        @pl.when(jnp.logical_or(jj < 29,
                                jnp.logical_and(jj == 29, wid == 0)))
        def _():
            fire_in(tB(jj + 1), winB, semIB)
        return cr
    lax.fori_loop(0, 31, step, 0)

    drain_out(tpA, semOA)

    @pl.when(wid == 0)
    def _():
        drain_out(tpB, semOB)

    @pl.when(wid == 0)
    def _():
        def trow(r, cr2):
            pltpu.sync_copy(tail16.at[r],
                            conv.at[pl.ds(NFULL * WIN * D + r * 128, 128)])
            return cr2
        lax.fori_loop(0, 16, trow, 0)


def _convert(out_t, emb_t, tail_o, tail_e):
    mesh = plsc.VectorSubcoreMesh(core_axis_name="c", subcore_axis_name="s")
    f = pl.kernel(
        _conv_body,
        out_type=[
            jax.ShapeDtypeStruct((V * D,), jnp.float32),
            jax.ShapeDtypeStruct((V * D,), jnp.float32),
        ],
        mesh=mesh,
        scratch_types=[
            pltpu.VMEM((D, WIN), jnp.float32),
            pltpu.VMEM((D, WIN), jnp.float32),
            pltpu.VMEM((WIN * D,), jnp.float32),
            pltpu.VMEM((WIN * D,), jnp.float32),
            pltpu.SemaphoreType.DMA,
            pltpu.SemaphoreType.DMA,
            pltpu.SemaphoreType.DMA,
            pltpu.SemaphoreType.DMA,
        ],
        compiler_params=pltpu.CompilerParams(needs_layout_passes=False,
                                             use_tc_tiling_on_sc=True),
    )
    return f(out_t, emb_t, tail_o, tail_e)


def _gather_body(conv, cemb, ctx, tgt, negt, pos_hbm, neg_hbm,
                 idxuA, idxuB, idxvA, idxvB, idxnA, idxnB, u_bufA, u_bufB,
                 v_bufA, v_bufB, vpA, vpB, posA, posB, negA, negB,
                 semXA, semXB, semGA, semGB, semWA, semWB):
    wid = lax.axis_index("s") * NC + lax.axis_index("c")
    iota = lax.iota(jnp.int32, L)

    def base(h):
        return wid * BPW + h * HG

    def fire_idx(h, idxu, idxv, idxn, sem):
        pltpu.async_copy(ctx.at[pl.ds(base(h), HG)], idxu, sem)
        pltpu.async_copy(tgt.at[pl.ds(base(h), HG)], idxv, sem)
        pltpu.async_copy(negt.at[:, pl.ds(base(h), HG)], idxn, sem)

    def drain_idx(idxu, idxv, idxn, sem):
        pltpu.make_async_copy(ctx.at[pl.ds(0, HG)], idxu, sem).wait()
        pltpu.make_async_copy(tgt.at[pl.ds(0, HG)], idxv, sem).wait()
        pltpu.make_async_copy(negt.at[:, pl.ds(0, HG)], idxn, sem).wait()

    def fire_g(idxu, idxv, idxn, u_buf, v_buf, vp, sem):
        pltpu.async_copy(cemb.at[idxu], u_buf, sem)
        pltpu.async_copy(conv.at[idxv], v_buf, sem)
        for j in range(K):
            pltpu.async_copy(conv.at[idxn.at[j]], vp.at[j], sem)

    def drain_g(u_buf, v_buf, vp, sem):
        pltpu.make_async_copy(cemb.at[idxuA], u_buf, sem).wait()
        pltpu.make_async_copy(conv.at[idxvA], v_buf, sem).wait()
        for j in range(K):
            pltpu.make_async_copy(conv.at[idxvA], vp.at[j], sem).wait()

    def compute(u_buf, v_buf, vp, pos_buf, neg_buf):
        for s in range(HG // L):
            rb = iota + (s * L)

            def dstep(d, accs):
                cols = jnp.full((L,), d, dtype=jnp.int32)
                u_col = plsc.load_gather(u_buf, [rb, cols])
                v_col = plsc.load_gather(v_buf, [rb, cols])
                new = [accs[0] + u_col * v_col]
                for j in range(K):
                    jj = jnp.full((L,), j, dtype=jnp.int32)
                    c = plsc.load_gather(vp, [jj, rb, cols])
                    new.append(accs[j + 1] + u_col * c)
                return new

            accs = lax.fori_loop(0, D, dstep,
                                 [jnp.zeros((L,), jnp.float32)] * (K + 1))
            pos_buf[pl.ds(s * L, L)] = accs[0]
            for j in range(K):
                neg_buf[j, pl.ds(s * L, L)] = accs[j + 1]

    def fire_out(h, pos_buf, neg_buf, sem):
        pltpu.async_copy(pos_buf, pos_hbm.at[pl.ds(base(h), HG)], sem)
        pltpu.async_copy(neg_buf, neg_hbm.at[wid * NHG + h], sem)

    def drain_out(pos_buf, neg_buf, sem):
        pltpu.make_async_copy(pos_buf, pos_hbm.at[pl.ds(0, HG)], sem).wait()
        pltpu.make_async_copy(neg_buf, neg_hbm.at[0], sem).wait()

    fire_idx(0, idxuA, idxvA, idxnA, semXA)
    drain_idx(idxuA, idxvA, idxnA, semXA)
    fire_g(idxuA, idxvA, idxnA, u_bufA, v_bufA, vpA, semGA)
    fire_idx(1, idxuB, idxvB, idxnB, semXB)

    def step(jj, cr):
        hA = 2 * jj
        hB = 2 * jj + 1
        drain_idx(idxuB, idxvB, idxnB, semXB)
        fire_g(idxuB, idxvB, idxnB, u_bufB, v_bufB, vpB, semGB)

        drain_g(u_bufA, v_bufA, vpA, semGA)
        compute(u_bufA, v_bufA, vpA, posA, negA)

        @pl.when(jj > 0)
        def _():
            drain_out(posA, negA, semWA)
        fire_out(hA, posA, negA, semWA)

        @pl.when(jj < 3)
        def _():
            fire_idx(hA + 2, idxuA, idxvA, idxnA, semXA)
            drain_idx(idxuA, idxvA, idxnA, semXA)
            fire_g(idxuA, idxvA, idxnA, u_bufA, v_bufA, vpA, semGA)
            fire_idx(hB + 2, idxuB, idxvB, idxnB, semXB)

        drain_g(u_bufB, v_bufB, vpB, semGB)
        compute(u_bufB, v_bufB, vpB, posB, negB)

        @pl.when(jj > 0)
        def _():
            drain_out(posB, negB, semWB)
        fire_out(hB, posB, negB, semWB)
        return cr
    lax.fori_loop(0, NHG // 2, step, 0)

    drain_out(posA, negA, semWA)
    drain_out(posB, negB, semWB)


def _gather(conv, cemb, ctx, tgt, negt):
    mesh = plsc.VectorSubcoreMesh(core_axis_name="c", subcore_axis_name="s")
    f = pl.kernel(
        _gather_body,
        out_type=[
            jax.ShapeDtypeStruct((B,), jnp.float32),
            jax.ShapeDtypeStruct((NW * NHG, K, HG), jnp.float32),
        ],
        mesh=mesh,
        scratch_types=[
            pltpu.VMEM((HG,), jnp.int32),
            pltpu.VMEM((HG,), jnp.int32),
            pltpu.VMEM((HG,), jnp.int32),
            pltpu.VMEM((HG,), jnp.int32),
            pltpu.VMEM((K, HG), jnp.int32),
            pltpu.VMEM((K, HG), jnp.int32),
            pltpu.VMEM((HG, D), jnp.float32),
            pltpu.VMEM((HG, D), jnp.float32),
            pltpu.VMEM((HG, D), jnp.float32),
            pltpu.VMEM((HG, D), jnp.float32),
            pltpu.VMEM((K, HG, D), jnp.float32),
            pltpu.VMEM((K, HG, D), jnp.float32),
            pltpu.VMEM((HG,), jnp.float32),
            pltpu.VMEM((HG,), jnp.float32),
            pltpu.VMEM((K, HG), jnp.float32),
            pltpu.VMEM((K, HG), jnp.float32),
            pltpu.SemaphoreType.DMA,
            pltpu.SemaphoreType.DMA,
            pltpu.SemaphoreType.DMA,
            pltpu.SemaphoreType.DMA,
            pltpu.SemaphoreType.DMA,
            pltpu.SemaphoreType.DMA,
        ],
        compiler_params=pltpu.CompilerParams(needs_layout_passes=False,
                                             use_tc_tiling_on_sc=False),
    )
    return f(conv, cemb, ctx, tgt, negt)


def _tc_loss_body(pos_ref, neg_ref, out_ref):
    p = pos_ref[...]
    n = neg_ref[...]

    def logsig(x):
        return jnp.minimum(x, 0.0) - jnp.log1p(jnp.exp(-jnp.abs(x)))

    total = jnp.sum(logsig(p)) + jnp.sum(logsig(-n))
    out_ref[...] = jnp.reshape(-total / B, (1, 1))


def _tc_loss(pos, neg):
    return pl.pallas_call(
        _tc_loss_body,
        out_shape=jax.ShapeDtypeStruct((1, 1), jnp.float32),
    )(pos.reshape(128, B // 128), neg.reshape(B * K // 128, 128))


def kernel(context, target, negatives, emb_table, out_table):
    out_t = out_table.T
    emb_t = emb_table.T
    neg_t = negatives.astype(jnp.int32).T
    tail_o = out_table[NFULL * WIN:, :].reshape(16, 128)
    tail_e = emb_table[NFULL * WIN:, :].reshape(16, 128)
    conv, cemb = _convert(out_t, emb_t, tail_o, tail_e)
    pos_sims, neg_sims = _gather(conv.reshape(V, D), cemb.reshape(V, D),
                                 context.astype(jnp.int32),
                                 target.astype(jnp.int32), neg_t)
    loss = _tc_loss(pos_sims, neg_sims.reshape(-1))
    return loss[0, 0]
